# baseline (speedup 1.0000x reference)
import functools

import jax
import jax.numpy as jnp
from jax import lax
from jax.experimental import pallas as pl
from jax.experimental.pallas import tpu as pltpu
from jax.experimental.pallas import tpu_sc as plsc

H = 128
_CHUNK = 128


def _sc_gather(emb, idx2d, B, D):
    info = plsc.get_sparse_core_info()
    NW = info.num_cores * info.num_subcores
    b_per_w = B // NW
    n_chunks = b_per_w // _CHUNK
    mesh = plsc.VectorSubcoreMesh(core_axis_name="c", subcore_axis_name="s")

    @functools.partial(
        pl.kernel,
        out_type=jax.ShapeDtypeStruct((B, D), jnp.float32),
        mesh=mesh,
        scratch_types=[
            pltpu.VMEM((n_chunks, _CHUNK), jnp.int32),
            pltpu.VMEM((b_per_w, D), jnp.float32),
            pltpu.SemaphoreType.DMA,
            pltpu.SemaphoreType.DMA,
        ],
    )
    def k(table_hbm, idx_hbm, out_hbm, idx_v, rows_v, sem, out_sem):
        wid = lax.axis_index("s") * info.num_cores + lax.axis_index("c")
        base = wid * b_per_w
        pltpu.sync_copy(idx_hbm.at[pl.ds(wid * n_chunks, n_chunks)], idx_v)
        gathers = []
        for j in range(n_chunks):
            gathers.append(
                pltpu.async_copy(
                    table_hbm.at[idx_v.at[j]],
                    rows_v.at[pl.ds(j * _CHUNK, _CHUNK)],
                    sem,
                )
            )
        scatters = []
        for j in range(n_chunks):
            gathers[j].wait()
            scatters.append(
                pltpu.async_copy(
                    rows_v.at[pl.ds(j * _CHUNK, _CHUNK)],
                    out_hbm.at[pl.ds(base + j * _CHUNK, _CHUNK)],
                    out_sem,
                )
            )
        for c in scatters:
            c.wait()

    return k(emb, idx2d)


_DN = (((1,), (1,)), ((), ()))


def _gru_body(x_ref, h_ref, wih_ref, whh_ref, bih_ref, bhh_ref, out_ref):
    x = x_ref[...]
    h = h_ref[...]
    gi = lax.dot_general(x, wih_ref[...], _DN, preferred_element_type=jnp.float32) + bih_ref[...]
    gh = lax.dot_general(h, whh_ref[...], _DN, preferred_element_type=jnp.float32) + bhh_ref[...]
    i_r, i_z, i_n = gi[:, :H], gi[:, H:2 * H], gi[:, 2 * H:]
    h_r, h_z, h_n = gh[:, :H], gh[:, H:2 * H], gh[:, 2 * H:]
    r = 0.5 + 0.5 * jnp.tanh(0.5 * (i_r + h_r))
    z = 0.5 + 0.5 * jnp.tanh(0.5 * (i_z + h_z))
    n = jnp.tanh(i_n + r * h_n)
    out_ref[...] = n + z * (h - n)


def _gru_body_alias(x_ref, h_ref, wih_ref, whh_ref, bih_ref, bhh_ref, y_ref, out_ref):
    del y_ref
    _gru_body(x_ref, h_ref, wih_ref, whh_ref, bih_ref, bhh_ref, out_ref)


def _tc_gru_half(x_half, h, wih, whh, bih, bhh, B, half, y_in=None, BB=4096):
    Bh = x_half.shape[0]
    nb = Bh // BB
    off = half * nb
    in_specs = [
        pl.BlockSpec((BB, H), lambda i: (i, 0)),
        pl.BlockSpec((BB, H), lambda i, off=off: (off + i, 0)),
        pl.BlockSpec((3 * H, H), lambda i: (0, 0)),
        pl.BlockSpec((3 * H, H), lambda i: (0, 0)),
        pl.BlockSpec((1, 3 * H), lambda i: (0, 0)),
        pl.BlockSpec((1, 3 * H), lambda i: (0, 0)),
    ]
    args = [x_half, h, wih, whh, bih, bhh]
    aliases = {}
    body = _gru_body
    if y_in is not None:
        in_specs.append(pl.BlockSpec(memory_space=pl.ANY))
        args.append(y_in)
        aliases = {6: 0}
        body = _gru_body_alias
    return pl.pallas_call(
        body,
        grid=(nb,),
        in_specs=in_specs,
        out_specs=pl.BlockSpec((BB, H), lambda i, off=off: (off + i, 0)),
        out_shape=jax.ShapeDtypeStruct((B, H), jnp.float32),
        input_output_aliases=aliases,
    )(*args)


def kernel(input, hidden, emb, W_ih, W_hh, b_ih, b_hh):
    B = input.shape[0]
    D = emb.shape[1]
    Bh = B // 2
    idx2d = input.reshape(B // _CHUNK, _CHUNK)
    nrows = Bh // _CHUNK
    x0 = _sc_gather(emb, idx2d[:nrows], Bh, D)
    x1 = _sc_gather(emb, idx2d[nrows:], Bh, D)
    bih = b_ih.reshape(1, 3 * H)
    bhh = b_hh.reshape(1, 3 * H)
    y0 = _tc_gru_half(x0, hidden, W_ih, W_hh, bih, bhh, B, half=0)
    return _tc_gru_half(x1, hidden, W_ih, W_hh, bih, bhh, B, half=1, y_in=y0)


# speedup vs baseline: 1.0249x; 1.0249x over previous
import functools

import jax
import jax.numpy as jnp
from jax import lax
from jax.experimental import pallas as pl
from jax.experimental.pallas import tpu as pltpu
from jax.experimental.pallas import tpu_sc as plsc

H = 128
_CHUNK = 128


def _sc_gather(emb, idx2d, B, D):
    info = plsc.get_sparse_core_info()
    NW = info.num_cores * info.num_subcores
    b_per_w = B // NW
    n_chunks = b_per_w // _CHUNK
    mesh = plsc.VectorSubcoreMesh(core_axis_name="c", subcore_axis_name="s")

    @functools.partial(
        pl.kernel,
        out_type=jax.ShapeDtypeStruct((B, D), jnp.float32),
        mesh=mesh,
        scratch_types=[
            pltpu.VMEM((n_chunks, _CHUNK), jnp.int32),
            pltpu.VMEM((b_per_w, D), jnp.float32),
            pltpu.SemaphoreType.DMA,
            pltpu.SemaphoreType.DMA,
        ],
    )
    def k(table_hbm, idx_hbm, out_hbm, idx_v, rows_v, sem, out_sem):
        wid = lax.axis_index("s") * info.num_cores + lax.axis_index("c")
        base = wid * b_per_w
        pltpu.sync_copy(idx_hbm.at[pl.ds(wid * n_chunks, n_chunks)], idx_v)
        gathers = []
        for j in range(n_chunks):
            gathers.append(
                pltpu.async_copy(
                    table_hbm.at[idx_v.at[j]],
                    rows_v.at[pl.ds(j * _CHUNK, _CHUNK)],
                    sem,
                )
            )
        scatters = []
        for j in range(n_chunks):
            gathers[j].wait()
            scatters.append(
                pltpu.async_copy(
                    rows_v.at[pl.ds(j * _CHUNK, _CHUNK)],
                    out_hbm.at[pl.ds(base + j * _CHUNK, _CHUNK)],
                    out_sem,
                )
            )
        for c in scatters:
            c.wait()

    return k(emb, idx2d)


_DN = (((1,), (1,)), ((), ()))


def _gru_body(x_ref, h_ref, wih_ref, whh_ref, bih_ref, bhh_ref, out_ref):
    x = x_ref[...]
    h = h_ref[...]
    xb = x.astype(jnp.bfloat16)
    hb = h.astype(jnp.bfloat16)
    gi = lax.dot_general(xb, wih_ref[...], _DN, preferred_element_type=jnp.float32) + bih_ref[...]
    gh = lax.dot_general(hb, whh_ref[...], _DN, preferred_element_type=jnp.float32) + bhh_ref[...]
    i_r, i_z, i_n = gi[:, :H], gi[:, H:2 * H], gi[:, 2 * H:]
    h_r, h_z, h_n = gh[:, :H], gh[:, H:2 * H], gh[:, 2 * H:]
    r = 0.5 + 0.5 * jnp.tanh(0.5 * (i_r + h_r))
    z = 0.5 + 0.5 * jnp.tanh(0.5 * (i_z + h_z))
    n = jnp.tanh(i_n + r * h_n)
    out_ref[...] = n + z * (h - n)


def _tc_gru(x, h, wih, whh, bih, bhh, B, BB=4096):
    grid = (B // BB,)
    return pl.pallas_call(
        _gru_body,
        grid=grid,
        in_specs=[
            pl.BlockSpec((BB, H), lambda i: (i, 0)),
            pl.BlockSpec((BB, H), lambda i: (i, 0)),
            pl.BlockSpec((3 * H, H), lambda i: (0, 0)),
            pl.BlockSpec((3 * H, H), lambda i: (0, 0)),
            pl.BlockSpec((1, 3 * H), lambda i: (0, 0)),
            pl.BlockSpec((1, 3 * H), lambda i: (0, 0)),
        ],
        out_specs=pl.BlockSpec((BB, H), lambda i: (i, 0)),
        out_shape=jax.ShapeDtypeStruct((B, H), jnp.float32),
    )(x, h, wih, whh, bih, bhh)


def kernel(input, hidden, emb, W_ih, W_hh, b_ih, b_hh):
    B = input.shape[0]
    D = emb.shape[1]
    idx2d = input.reshape(B // _CHUNK, _CHUNK)
    x = _sc_gather(emb, idx2d, B, D)
    return _tc_gru(
        x,
        hidden,
        W_ih.astype(jnp.bfloat16),
        W_hh.astype(jnp.bfloat16),
        b_ih.reshape(1, 3 * H),
        b_hh.reshape(1, 3 * H),
        B,
    )
